# baseline (speedup 1.0000x reference)
import functools

import jax
import jax.numpy as jnp
from jax import lax
from jax.experimental import pallas as pl
from jax.experimental.pallas import tpu as pltpu
from jax.experimental.pallas import tpu_sc as plsc

F32 = jnp.float32
NHEAD = 8
DH = 8
LD = 64
WROW = 128
NBIN = 10
NLAYER = 2
NW = 32
CH = 128
CHG = 128
RACC = 10240
RPT = RACC // 16
VROW = 72
EBLK = 4096
NBLK = 1000

_mesh = functools.partial(
    plsc.VectorSubcoreMesh, core_axis_name="c", subcore_axis_name="s",
    num_cores=2, num_subcores=16)


def _pad_rows(n, q):
    m = -(-n // q) * q
    while m % EBLK:
        m += q
    return m


def _sc_gather_sum(tables, idxs, npad):
    ntab = len(tables)
    per_tile = npad // NW
    nch = per_tile // CHG
    NB = 4
    assert nch % NB == 0

    @functools.partial(
        pl.kernel,
        out_type=jax.ShapeDtypeStruct((npad, WROW), F32),
        mesh=_mesh(),
        scratch_types=(
            [pltpu.VMEM((CHG,), jnp.int32) for _ in range(NB * ntab)]
            + [pltpu.VMEM((CHG, WROW), F32) for _ in range(NB)]
            + [pltpu.SemaphoreType.DMA for _ in range(2 * NB)]
        ),
    )
    def k(*refs):
        tabs = refs[:ntab]
        idx = refs[ntab:2 * ntab]
        out = refs[2 * ntab]
        sc = refs[2 * ntab + 1:]
        ivs = [sc[b * ntab:(b + 1) * ntab] for b in range(NB)]
        bufs = sc[NB * ntab:NB * ntab + NB]
        sg = sc[NB * ntab + NB:NB * ntab + 2 * NB]
        ss = sc[NB * ntab + 2 * NB:NB * ntab + 3 * NB]
        wid = lax.axis_index("s") * 2 + lax.axis_index("c")
        base0 = wid * per_tile

        def fire_idx(ci, b):
            base = base0 + ci * CHG
            for j in range(ntab):
                pltpu.async_copy(idx[j].at[pl.ds(base, CHG)], ivs[b][j], sg[b])

        for b in range(NB):
            fire_idx(b, b)

        def body(k4, carry):
            for b in range(NB):
                ci = NB * k4 + b
                base = base0 + ci * CHG
                for j in range(ntab):
                    pltpu.make_async_copy(
                        idx[j].at[pl.ds(base, CHG)], ivs[b][j], sg[b]).wait()

                @pl.when(ci >= NB)
                def _():
                    pltpu.make_async_copy(
                        bufs[b], out.at[pl.ds(base - NB * CHG, CHG)],
                        ss[b]).wait()

                pltpu.async_copy(tabs[0].at[ivs[b][0]], bufs[b], sg[b])
            for b in range(NB):
                pltpu.make_async_copy(
                    tabs[0].at[ivs[b][0]], bufs[b], sg[b]).wait()
                for j in range(1, ntab):
                    pltpu.async_copy(tabs[j].at[ivs[b][j]], bufs[b], sg[b],
                                     add=True)
            for b in range(NB):
                ci = NB * k4 + b
                base = base0 + ci * CHG
                for j in range(1, ntab):
                    pltpu.make_async_copy(
                        tabs[j].at[ivs[b][j]], bufs[b], sg[b]).wait()
                pltpu.async_copy(bufs[b], out.at[pl.ds(base, CHG)], ss[b])

                @pl.when(ci + NB < nch)
                def _():
                    fire_idx(ci + NB, b)
            return carry

        lax.fori_loop(0, nch // NB, body, 0)
        for b in range(NB):
            base_l = base0 + (nch - NB + b) * CHG
            pltpu.make_async_copy(
                bufs[b], out.at[pl.ds(base_l, CHG)], ss[b]).wait()

    return k(*tables, *idxs)


def _sc_scatter(tidx, vals, npad):
    per_tile = npad // NW
    nch = per_tile // CH
    z = jnp.zeros((RACC, VROW), F32)

    assert nch % 2 == 0

    @functools.partial(
        pl.kernel,
        out_type=jax.ShapeDtypeStruct((2, RACC, VROW), F32),
        mesh=_mesh(),
        scratch_types=(
            [pltpu.VMEM((CH,), jnp.int32) for _ in range(2)]
            + [pltpu.VMEM((CH, VROW), F32) for _ in range(2)]
            + [pltpu.VMEM_SHARED((RACC, VROW), F32)]
            + [pltpu.SemaphoreType.DMA for _ in range(4)]
        ),
    )
    def k(ti, vv, zz, out, tv0, tv1, b0, b1, acc, sl0, sl1, sc0, sc1):
        tvs, bufs, sl, sc = (tv0, tv1), (b0, b1), (sl0, sl1), (sc0, sc1)
        cid = lax.axis_index("c")
        sid = lax.axis_index("s")
        r0 = sid * RPT
        pltpu.sync_copy(zz.at[pl.ds(r0, RPT)], acc.at[pl.ds(r0, RPT)])
        plsc.subcore_barrier()
        wid = sid * 2 + cid
        base0 = wid * per_tile

        def body(k2, carry):
            for b in range(2):
                ci = 2 * k2 + b
                base = base0 + ci * CH

                @pl.when(ci >= 2)
                def _():
                    pltpu.make_async_copy(
                        bufs[b], acc.at[tvs[b]], sc[b]).wait()

                pltpu.async_copy(ti.at[pl.ds(base, CH)], tvs[b], sl[b])
                pltpu.async_copy(vv.at[pl.ds(base, CH)], bufs[b], sl[b])
                pltpu.make_async_copy(
                    ti.at[pl.ds(base, CH)], tvs[b], sl[b]).wait()
                pltpu.make_async_copy(
                    vv.at[pl.ds(base, CH)], bufs[b], sl[b]).wait()
                pltpu.async_copy(bufs[b], acc.at[tvs[b]], sc[b], add=True)
            return carry

        lax.fori_loop(0, nch // 2, body, 0)
        for b in range(2):
            pltpu.make_async_copy(bufs[b], acc.at[tvs[b]], sc[b]).wait()
        plsc.subcore_barrier()
        pltpu.sync_copy(acc.at[pl.ds(r0, RPT)], out.at[cid, pl.ds(r0, RPT)])

    return k(tidx, vals, z)


def _sc_gather_scatter(table, ridx, tidx, npad):
    per_tile = npad // NW
    nch = per_tile // CH
    z = jnp.zeros((RACC, WROW), F32)

    assert nch % 2 == 0

    @functools.partial(
        pl.kernel,
        out_type=jax.ShapeDtypeStruct((2, RACC, WROW), F32),
        mesh=_mesh(),
        scratch_types=(
            [pltpu.VMEM((CH,), jnp.int32) for _ in range(4)]
            + [pltpu.VMEM((CH, WROW), F32) for _ in range(2)]
            + [pltpu.VMEM_SHARED((RACC, WROW), F32)]
            + [pltpu.SemaphoreType.DMA for _ in range(4)]
        ),
    )
    def k(tab, ri, ti, zz, out, rv0, rv1, tv0, tv1, b0, b1, acc,
          sg0, sg1, sc0, sc1):
        rvs, tvs, bufs = (rv0, rv1), (tv0, tv1), (b0, b1)
        sg, sc = (sg0, sg1), (sc0, sc1)
        cid = lax.axis_index("c")
        sid = lax.axis_index("s")
        r0 = sid * RPT
        pltpu.sync_copy(zz.at[pl.ds(r0, RPT)], acc.at[pl.ds(r0, RPT)])
        plsc.subcore_barrier()
        wid = sid * 2 + cid
        base0 = wid * per_tile

        def body(k2, carry):
            for b in range(2):
                ci = 2 * k2 + b
                base = base0 + ci * CH

                @pl.when(ci >= 2)
                def _():
                    pltpu.make_async_copy(
                        bufs[b], acc.at[tvs[b]], sc[b]).wait()

                pltpu.async_copy(ri.at[pl.ds(base, CH)], rvs[b], sg[b])
                pltpu.async_copy(ti.at[pl.ds(base, CH)], tvs[b], sg[b])
                pltpu.make_async_copy(
                    ri.at[pl.ds(base, CH)], rvs[b], sg[b]).wait()
                pltpu.make_async_copy(
                    ti.at[pl.ds(base, CH)], tvs[b], sg[b]).wait()
                pltpu.async_copy(tab.at[rvs[b]], bufs[b], sg[b])
            for b in range(2):
                ci = 2 * k2 + b
                pltpu.make_async_copy(
                    tab.at[rvs[b]], bufs[b], sg[b]).wait()
                pltpu.async_copy(bufs[b], acc.at[tvs[b]], sc[b], add=True)
            return carry

        lax.fori_loop(0, nch // 2, body, 0)
        for b in range(2):
            pltpu.make_async_copy(bufs[b], acc.at[tvs[b]], sc[b]).wait()
        plsc.subcore_barrier()
        pltpu.sync_copy(acc.at[pl.ds(r0, RPT)], out.at[cid, pl.ds(r0, RPT)])

    return k(table, ridx, tidx, z)


def _tc_hist(h8, tb8):
    npad = h8.shape[0]
    grid = npad // EBLK

    def body(hr, tr, out):
        oh = (hr[:, 0:1] == lax.broadcasted_iota(jnp.int32, (EBLK, 16), 1)
              ).astype(F32)
        otb = (tr[:, 0:1] == lax.broadcasted_iota(jnp.int32, (EBLK, WROW), 1)
               ).astype(F32)
        part = lax.dot_general(oh, otb, (((0,), (0,)), ((), ())),
                               preferred_element_type=F32)

        @pl.when(pl.program_id(0) == 0)
        def _():
            out[...] = jnp.zeros((16, WROW), F32)

        out[...] += part

    return pl.pallas_call(
        body, grid=(grid,),
        in_specs=[pl.BlockSpec((EBLK, 8), lambda i: (i, 0))] * 2,
        out_specs=_full((16, WROW)),
        out_shape=jax.ShapeDtypeStruct((16, WROW), F32),
    )(h8, tb8)


def _dot(a, b):
    return jnp.dot(a, b, preferred_element_type=F32)


def _lrelu(x):
    return jnp.maximum(x, 0.2 * x)


def _full(shape):
    return pl.BlockSpec(shape, lambda i: tuple(0 for _ in shape))


def _bc8(v, m):
    return jnp.broadcast_to(v.reshape(1, m), (8, m))


def _lin(x, wT, b=None, add=None, relu=False):
    n, kdim = x.shape
    m = wT.shape[1]
    grid = n // NBLK
    in_specs = [pl.BlockSpec((NBLK, kdim), lambda i: (i, 0)),
                _full((kdim, m))]
    args = [x, wT]
    if b is not None:
        in_specs.append(_full((8, m)))
        args.append(_bc8(b, m))
    if add is not None:
        in_specs.append(pl.BlockSpec((NBLK, m), lambda i: (i, 0)))
        args.append(add)

    def body(*refs):
        y = _dot(refs[0][...], refs[1][...])
        idx = 2
        if b is not None:
            y = y + refs[idx][0:1, :]
            idx += 1
        if add is not None:
            y = y + refs[idx][...]
            idx += 1
        if relu:
            y = jnp.maximum(y, 0.0)
        refs[-1][...] = y

    return pl.pallas_call(
        body, grid=(grid,), in_specs=in_specs,
        out_specs=pl.BlockSpec((NBLK, m), lambda i: (i, 0)),
        out_shape=jax.ShapeDtypeStruct((n, m), F32),
    )(*args)


def _ent_tables(le, wT, b):
    n = le.shape[0]
    grid = n // NBLK

    def body(xr, wr, br, o1, o2):
        y = _dot(xr[...], wr[...]) + br[0:1, :]
        o1[...] = jnp.concatenate(
            [y[:, :LD], jnp.zeros((NBLK, LD), F32)], axis=1)
        o2[...] = y[:, LD:]

    return pl.pallas_call(
        body, grid=(grid,),
        in_specs=[pl.BlockSpec((NBLK, LD), lambda i: (i, 0)),
                  _full((LD, 3 * LD)), _full((8, 3 * LD))],
        out_specs=[pl.BlockSpec((NBLK, WROW), lambda i: (i, 0))] * 2,
        out_shape=[jax.ShapeDtypeStruct((n, WROW), F32)] * 2,
    )(le, wT, _bc8(b, 3 * LD))


def _edge_raw(gath, v64, S):
    npad = gath.shape[0]
    grid = npad // EBLK

    def body(pr, vr, sr, raw, pmax):
        h = _lrelu(pr[:, :LD]) * vr[0:1, :]
        r = _dot(h, sr[...])
        raw[...] = r
        pmax[...] = jnp.max(r, axis=0, keepdims=True)[None]

    return pl.pallas_call(
        body, grid=(grid,),
        in_specs=[pl.BlockSpec((EBLK, WROW), lambda i: (i, 0)),
                  _full((8, LD)), _full((LD, NHEAD))],
        out_specs=[pl.BlockSpec((EBLK, NHEAD), lambda i: (i, 0)),
                   pl.BlockSpec((1, 1, NHEAD), lambda i: (i, 0, 0))],
        out_shape=[jax.ShapeDtypeStruct((npad, NHEAD), F32),
                   jax.ShapeDtypeStruct((grid, 1, NHEAD), F32)],
    )(gath, _bc8(v64, LD), S)


def _edge_scale(raw, gath, gmax8, Sexp):
    npad = raw.shape[0]
    grid = npad // EBLK

    def body(rr, vr, gr, er, out):
        a = jnp.exp(rr[...] - gr[0:1, :])
        aggr = _dot(a, er[...]) * vr[:, LD:]
        out[...] = jnp.concatenate([aggr, a], axis=1)

    return pl.pallas_call(
        body, grid=(grid,),
        in_specs=[pl.BlockSpec((EBLK, NHEAD), lambda i: (i, 0)),
                  pl.BlockSpec((EBLK, WROW), lambda i: (i, 0)),
                  _full((8, NHEAD)), _full((NHEAD, LD))],
        out_specs=pl.BlockSpec((EBLK, VROW), lambda i: (i, 0)),
        out_shape=jax.ShapeDtypeStruct((npad, VROW), F32),
    )(raw, gath, gmax8, Sexp)


def _self_div(a0, a1):
    def body(r0, r1, out):
        s = r0[...] + r1[...]
        out[...] = s[:, :LD] / (s[:, LD:LD + 1] + 1e-16)

    return pl.pallas_call(
        body, grid=(10,),
        in_specs=[pl.BlockSpec((NBLK, WROW), lambda i: (i, 0))] * 2,
        out_specs=pl.BlockSpec((NBLK, LD), lambda i: (i, 0)),
        out_shape=jax.ShapeDtypeStruct((10000, LD), F32),
    )(a0, a1)


def _ent_combine(Tt, Th, CsHs, o0, le, WresT, bres, v64, S, Sexp, gmax8):
    nacc = len(o0)

    def body(*refs):
        ttr, thr, chr_ = refs[0], refs[1], refs[2]
        oas = refs[3:3 + nacc]
        ler, wr, brr, vr, sr, er, gmr, out = refs[3 + nacc:]
        A = ttr[:, :LD]
        B = thr[:, :LD]
        G = thr[:, LD:]
        cs = chr_[:, :LD]
        hs = chr_[:, LD:]
        h = _lrelu(A + B + cs) * vr[0:1, :]
        raw_s = _dot(h, sr[...])
        attn_s = jnp.exp(raw_s - gmr[0:1, :])
        vs = G + hs
        acc = oas[0][...]
        for oa in oas[1:]:
            acc = acc + oa[...]
        den = acc[:, LD:LD + NHEAD] + attn_s
        num = acc[:, :LD] + _dot(attn_s, er[...]) * vs
        o = num / (_dot(den, er[...]) + 1e-38)
        o = o + _dot(ler[...], wr[...]) + brr[0:1, :]
        out[...] = jnp.maximum(o, 0.0)

    blk = lambda w: pl.BlockSpec((NBLK, w), lambda i: (i, 0))
    return pl.pallas_call(
        body, grid=(10,),
        in_specs=[blk(WROW), blk(WROW), blk(WROW)]
        + [blk(VROW)] * len(o0)
        + [blk(LD), _full((LD, LD)), _full((8, LD)), _full((8, LD)),
           _full((LD, NHEAD)), _full((NHEAD, LD)), _full((8, NHEAD))],
        out_specs=blk(LD),
        out_shape=jax.ShapeDtypeStruct((10000, LD), F32),
    )(Tt, Th, CsHs, *o0, le, WresT, _bc8(bres, LD), _bc8(v64, LD),
      S, Sexp, gmax8)


def _rel_dense(lr16, W1hT, W1tT, battn, WaT, ba, bin16, n8, Oh, Ot, Ob,
               v64, S, Sexp):
    def body(lrr, whr, wtr, bar, war, bagr, binr, n8r, ohr, otr, obr,
             vr, sr, er, out):
        lrv = lrr[...]
        Ah = _dot(lrv, whr[...]) + bar[0:1, :]
        Bt = _dot(lrv, wtr[...])
        V = _dot(lrv, war[...]) + bagr[0:1, :]
        n1 = n8r[:, 0:1]
        pre = _dot(ohr[...], Ah) + _dot(otr[...], Bt)
        raw = _dot(_lrelu(pre) * vr[0:1, :], sr[...]) + _dot(obr[...], binr[...])
        gmax = jnp.max(raw, axis=0, keepdims=True)
        E = n1 * jnp.exp(raw - gmax)
        dnums = (((0,), (0,)), ((), ()))
        den = lax.dot_general(ohr[...], E, dnums, preferred_element_type=F32)
        Vc = _dot(otr[...], V)
        num = lax.dot_general(ohr[...], _dot(E, er[...]) * Vc, dnums,
                              preferred_element_type=F32)
        out[...] = num / (_dot(den, er[...]) + 1e-38)

    return pl.pallas_call(
        body, grid=(1,),
        in_specs=[_full((16, LD)), _full((LD, LD)), _full((LD, LD)),
                  _full((8, LD)), _full((LD, LD)), _full((8, LD)),
                  _full((16, NHEAD)),
                  _full((1024, 8)),
                  _full((1024, 16)), _full((1024, 16)), _full((1024, 16)),
                  _full((8, LD)), _full((LD, NHEAD)), _full((NHEAD, LD))],
        out_specs=_full((16, LD)),
        out_shape=jax.ShapeDtypeStruct((16, LD), F32),
    )(lr16, W1hT, W1tT, _bc8(battn, LD), WaT, _bc8(ba, LD), bin16, n8,
      Oh, Ot, Ob, _bc8(v64, LD), S, Sexp)


def kernel(emb_ent, emb_rel, triplets, relation_triplets, params):
    S = (jnp.arange(LD)[:, None] // DH == jnp.arange(NHEAD)[None, :]).astype(F32)
    Sexp = S.T

    n_tri = triplets.shape[0]
    npad_e = _pad_rows(n_tri, NW * CHG * 4)
    pe = npad_e - n_tri
    h_g = jnp.pad(triplets[:, 0], (0, pe))
    r_g = jnp.pad(triplets[:, 1], (0, pe))
    t_g = jnp.pad(triplets[:, 2], (0, pe))
    t_s = jnp.pad(triplets[:, 2], (0, pe), constant_values=10000)

    n_rt = relation_triplets.shape[0]
    npad_r = _pad_rows(n_rt, EBLK)
    cidx = (relation_triplets[:, 0] * 100 + relation_triplets[:, 1] * 10
            + relation_triplets[:, 2])
    cidx = jnp.pad(cidx, (0, npad_r - n_rt), constant_values=1000)
    h8 = jnp.broadcast_to((cidx // 100)[:, None], (npad_r, 8))
    tb8 = jnp.broadcast_to((cidx % 100)[:, None], (npad_r, 8))

    c1024 = jnp.arange(1024)
    valid = (c1024 < 1000)[:, None]
    Oh = ((c1024[:, None] // 100 == jnp.arange(16)[None, :]) & valid).astype(F32)
    Ot = (((c1024[:, None] // 10) % 10 == jnp.arange(16)[None, :]) & valid).astype(F32)
    Ob = ((c1024[:, None] % 10 == jnp.arange(16)[None, :]) & valid).astype(F32)

    le = _lin(emb_ent, params['ent_proj1_w'].T, params['ent_proj1_b'])
    lr = _lin(emb_rel, params['rel_proj1_w'].T, params['rel_proj1_b'])

    N16 = _tc_hist(h8, tb8)
    n1024 = jnp.pad(N16[:10, :100].reshape(-1), (0, 24))
    n8 = jnp.broadcast_to(n1024[:, None], (1024, 8))
    for i in range(NLAYER):
        p = params['rel_layers'][i]
        W = p['attn_proj_w']
        lr16 = jnp.pad(lr[:10], ((0, 6), (0, 0)))
        bin16 = jnp.pad(p['attn_bin'].reshape(NBIN, NHEAD), ((0, 6), (0, 0)))
        out16 = _rel_dense(lr16, W[:, :LD].T, W[:, LD:].T, p['attn_proj_b'],
                           p['aggr_proj_w'].T, p['aggr_proj_b'], bin16,
                           n8, Oh, Ot, Ob, p['attn_vec'], S, Sexp)
        out_full = jnp.zeros((10000, LD), F32).at[:10].set(out16[:10])
        pr = params['res_rel'][i]
        lr = _lin(lr, pr['w'].T, pr['b'], add=out_full, relu=True)

    table = jnp.concatenate(
        [lr, jnp.ones((10000, 1), F32), jnp.zeros((10000, WROW - LD - 1), F32)],
        axis=1)
    sacc = _sc_gather_scatter(table, r_g, t_s, npad_e)
    self_rel = _self_div(sacc[0, :10000], sacc[1, :10000])

    for i in range(NLAYER):
        p = params['ent_layers'][i]
        W = p['attn_proj_w']
        Wa = p['aggr_proj_w']
        bb = jnp.concatenate([p['attn_proj_b'], jnp.zeros((LD,), F32),
                              p['aggr_proj_b']])
        wcat = jnp.concatenate(
            [W[:, :LD].T, W[:, LD:2 * LD].T, Wa[:, :LD].T], axis=1)
        Tt, Th = _ent_tables(le, wcat, bb)
        wr_cat = jnp.concatenate([W[:, 2 * LD:].T, Wa[:, LD:].T], axis=1)
        Tr = _lin(lr, wr_cat)
        CsHs = _lin(self_rel, wr_cat)

        gath = _sc_gather_sum([Tt, Th, Tr], [t_g, h_g, r_g], npad_e)
        raw, pmax = _edge_raw(gath, p['attn_vec'], S)
        gmax8 = jnp.broadcast_to(
            jnp.max(pmax, axis=(0, 1)).reshape(1, NHEAD), (8, NHEAD))
        vals = _edge_scale(raw, gath, gmax8, Sexp)
        oacc = _sc_scatter(t_s, vals, npad_e)
        oaccs = [oacc[0, :10000], oacc[1, :10000]]

        pr = params['res_ent'][i]
        le = _ent_combine(Tt, Th, CsHs, oaccs, le, pr['w'].T, pr['b'],
                          p['attn_vec'], S, Sexp, gmax8)

    out_ent = _lin(le, params['ent_proj2_w'].T, params['ent_proj2_b'])
    out_rel = _lin(lr, params['rel_proj2_w'].T, params['rel_proj2_b'])
    return out_ent, out_rel


# speedup vs baseline: 1.4537x; 1.4537x over previous
import functools

import jax
import jax.numpy as jnp
from jax import lax
from jax.experimental import pallas as pl
from jax.experimental.pallas import tpu as pltpu
from jax.experimental.pallas import tpu_sc as plsc

F32 = jnp.float32
NHEAD = 8
DH = 8
LD = 64
WROW = 128
NBIN = 10
NLAYER = 2
NW = 32
CH = 128
CHG = 128
RACC = 10240
RPT = RACC // 16
VROW = 72
EBLK = 4096
NBLK = 1000

_mesh = functools.partial(
    plsc.VectorSubcoreMesh, core_axis_name="c", subcore_axis_name="s",
    num_cores=2, num_subcores=16)


def _pad_rows(n, q):
    m = -(-n // q) * q
    while m % EBLK:
        m += q
    return m


def _sc_gather_sum(tables, idxs, npad):
    ntab = len(tables)
    per_tile = npad // NW
    nch = per_tile // CHG
    assert nch % 2 == 0

    @functools.partial(
        pl.kernel,
        out_type=jax.ShapeDtypeStruct((npad, WROW), F32),
        mesh=_mesh(),
        scratch_types=(
            [pltpu.VMEM((CHG,), jnp.int32) for _ in range(2 * ntab)]
            + [pltpu.VMEM((CHG, WROW), F32) for _ in range(2)]
            + [pltpu.SemaphoreType.DMA for _ in range(6)]
        ),
    )
    def k(*refs):
        tabs = refs[:ntab]
        idx = refs[ntab:2 * ntab]
        out = refs[2 * ntab]
        sc = refs[2 * ntab + 1:]
        ivs = [sc[:ntab], sc[ntab:2 * ntab]]
        bufs = sc[2 * ntab:2 * ntab + 2]
        sg = sc[2 * ntab + 2:2 * ntab + 4]
        ss = sc[2 * ntab + 4:2 * ntab + 6]
        wid = lax.axis_index("s") * 2 + lax.axis_index("c")
        base0 = wid * per_tile

        def fire_idx(ci, b):
            base = base0 + ci * CHG
            for j in range(ntab):
                pltpu.async_copy(idx[j].at[pl.ds(base, CHG)], ivs[b][j], sg[b])

        fire_idx(0, 0)
        fire_idx(1, 1)

        def body(k2, carry):
            for b in range(2):
                ci = 2 * k2 + b
                base = base0 + ci * CHG
                for j in range(ntab):
                    pltpu.make_async_copy(
                        idx[j].at[pl.ds(base, CHG)], ivs[b][j], sg[b]).wait()

                @pl.when(ci >= 2)
                def _():
                    pltpu.make_async_copy(
                        bufs[b], out.at[pl.ds(base - 2 * CHG, CHG)],
                        ss[b]).wait()

                pltpu.async_copy(tabs[0].at[ivs[b][0]], bufs[b], sg[b])
            for b in range(2):
                pltpu.make_async_copy(
                    tabs[0].at[ivs[b][0]], bufs[b], sg[b]).wait()
                for j in range(1, ntab):
                    pltpu.async_copy(tabs[j].at[ivs[b][j]], bufs[b], sg[b],
                                     add=True)
            for b in range(2):
                ci = 2 * k2 + b
                base = base0 + ci * CHG
                for j in range(1, ntab):
                    pltpu.make_async_copy(
                        tabs[j].at[ivs[b][j]], bufs[b], sg[b]).wait()
                pltpu.async_copy(bufs[b], out.at[pl.ds(base, CHG)], ss[b])

                @pl.when(ci + 2 < nch)
                def _():
                    fire_idx(ci + 2, b)
            return carry

        lax.fori_loop(0, nch // 2, body, 0)
        for b in range(2):
            base_l = base0 + (nch - 2 + b) * CHG
            pltpu.make_async_copy(
                bufs[b], out.at[pl.ds(base_l, CHG)], ss[b]).wait()

    return k(*tables, *idxs)


def _sc_scatter(tidx, vals, npad):
    per_tile = npad // NW
    nch = per_tile // CH
    z = jnp.zeros((RACC, VROW), F32)

    assert nch % 2 == 0

    @functools.partial(
        pl.kernel,
        out_type=jax.ShapeDtypeStruct((2, RACC, VROW), F32),
        mesh=_mesh(),
        scratch_types=(
            [pltpu.VMEM((CH,), jnp.int32) for _ in range(2)]
            + [pltpu.VMEM((CH, VROW), F32) for _ in range(2)]
            + [pltpu.VMEM_SHARED((RACC, VROW), F32)]
            + [pltpu.SemaphoreType.DMA for _ in range(4)]
        ),
    )
    def k(ti, vv, zz, out, tv0, tv1, b0, b1, acc, sl0, sl1, sc0, sc1):
        tvs, bufs, sl, sc = (tv0, tv1), (b0, b1), (sl0, sl1), (sc0, sc1)
        cid = lax.axis_index("c")
        sid = lax.axis_index("s")
        r0 = sid * RPT
        pltpu.sync_copy(zz.at[pl.ds(r0, RPT)], acc.at[pl.ds(r0, RPT)])
        plsc.subcore_barrier()
        wid = sid * 2 + cid
        base0 = wid * per_tile

        def body(k2, carry):
            for b in range(2):
                ci = 2 * k2 + b
                base = base0 + ci * CH

                @pl.when(ci >= 2)
                def _():
                    pltpu.make_async_copy(
                        bufs[b], acc.at[tvs[b]], sc[b]).wait()

                pltpu.async_copy(ti.at[pl.ds(base, CH)], tvs[b], sl[b])
                pltpu.async_copy(vv.at[pl.ds(base, CH)], bufs[b], sl[b])
                pltpu.make_async_copy(
                    ti.at[pl.ds(base, CH)], tvs[b], sl[b]).wait()
                pltpu.make_async_copy(
                    vv.at[pl.ds(base, CH)], bufs[b], sl[b]).wait()
                pltpu.async_copy(bufs[b], acc.at[tvs[b]], sc[b], add=True)
            return carry

        lax.fori_loop(0, nch // 2, body, 0)
        for b in range(2):
            pltpu.make_async_copy(bufs[b], acc.at[tvs[b]], sc[b]).wait()
        plsc.subcore_barrier()
        pltpu.sync_copy(acc.at[pl.ds(r0, RPT)], out.at[cid, pl.ds(r0, RPT)])

    return k(tidx, vals, z)


def _sc_gather_scatter(table, ridx, tidx, npad):
    per_tile = npad // NW
    nch = per_tile // CH
    z = jnp.zeros((RACC, WROW), F32)

    assert nch % 2 == 0

    @functools.partial(
        pl.kernel,
        out_type=jax.ShapeDtypeStruct((2, RACC, WROW), F32),
        mesh=_mesh(),
        scratch_types=(
            [pltpu.VMEM((CH,), jnp.int32) for _ in range(4)]
            + [pltpu.VMEM((CH, WROW), F32) for _ in range(2)]
            + [pltpu.VMEM_SHARED((RACC, WROW), F32)]
            + [pltpu.SemaphoreType.DMA for _ in range(4)]
        ),
    )
    def k(tab, ri, ti, zz, out, rv0, rv1, tv0, tv1, b0, b1, acc,
          sg0, sg1, sc0, sc1):
        rvs, tvs, bufs = (rv0, rv1), (tv0, tv1), (b0, b1)
        sg, sc = (sg0, sg1), (sc0, sc1)
        cid = lax.axis_index("c")
        sid = lax.axis_index("s")
        r0 = sid * RPT
        pltpu.sync_copy(zz.at[pl.ds(r0, RPT)], acc.at[pl.ds(r0, RPT)])
        plsc.subcore_barrier()
        wid = sid * 2 + cid
        base0 = wid * per_tile

        def body(k2, carry):
            for b in range(2):
                ci = 2 * k2 + b
                base = base0 + ci * CH

                @pl.when(ci >= 2)
                def _():
                    pltpu.make_async_copy(
                        bufs[b], acc.at[tvs[b]], sc[b]).wait()

                pltpu.async_copy(ri.at[pl.ds(base, CH)], rvs[b], sg[b])
                pltpu.async_copy(ti.at[pl.ds(base, CH)], tvs[b], sg[b])
                pltpu.make_async_copy(
                    ri.at[pl.ds(base, CH)], rvs[b], sg[b]).wait()
                pltpu.make_async_copy(
                    ti.at[pl.ds(base, CH)], tvs[b], sg[b]).wait()
                pltpu.async_copy(tab.at[rvs[b]], bufs[b], sg[b])
            for b in range(2):
                ci = 2 * k2 + b
                pltpu.make_async_copy(
                    tab.at[rvs[b]], bufs[b], sg[b]).wait()
                pltpu.async_copy(bufs[b], acc.at[tvs[b]], sc[b], add=True)
            return carry

        lax.fori_loop(0, nch // 2, body, 0)
        for b in range(2):
            pltpu.make_async_copy(bufs[b], acc.at[tvs[b]], sc[b]).wait()
        plsc.subcore_barrier()
        pltpu.sync_copy(acc.at[pl.ds(r0, RPT)], out.at[cid, pl.ds(r0, RPT)])

    return k(table, ridx, tidx, z)


def _tc_hist(h8, tb8):
    npad = h8.shape[0]
    grid = npad // EBLK

    def body(hr, tr, out):
        oh = (hr[:, 0:1] == lax.broadcasted_iota(jnp.int32, (EBLK, 16), 1)
              ).astype(F32)
        otb = (tr[:, 0:1] == lax.broadcasted_iota(jnp.int32, (EBLK, WROW), 1)
               ).astype(F32)
        part = lax.dot_general(oh, otb, (((0,), (0,)), ((), ())),
                               preferred_element_type=F32)

        @pl.when(pl.program_id(0) == 0)
        def _():
            out[...] = jnp.zeros((16, WROW), F32)

        out[...] += part

    return pl.pallas_call(
        body, grid=(grid,),
        in_specs=[pl.BlockSpec((EBLK, 8), lambda i: (i, 0))] * 2,
        out_specs=_full((16, WROW)),
        out_shape=jax.ShapeDtypeStruct((16, WROW), F32),
    )(h8, tb8)


def _dot(a, b):
    return jnp.dot(a, b, preferred_element_type=F32)


def _lrelu(x):
    return jnp.maximum(x, 0.2 * x)


def _full(shape):
    return pl.BlockSpec(shape, lambda i: tuple(0 for _ in shape))


def _bc8(v, m):
    return jnp.broadcast_to(v.reshape(1, m), (8, m))


def _lin(x, wT, b=None, add=None, relu=False):
    n, kdim = x.shape
    m = wT.shape[1]
    grid = n // NBLK
    in_specs = [pl.BlockSpec((NBLK, kdim), lambda i: (i, 0)),
                _full((kdim, m))]
    args = [x, wT]
    if b is not None:
        in_specs.append(_full((8, m)))
        args.append(_bc8(b, m))
    if add is not None:
        in_specs.append(pl.BlockSpec((NBLK, m), lambda i: (i, 0)))
        args.append(add)

    def body(*refs):
        y = _dot(refs[0][...], refs[1][...])
        idx = 2
        if b is not None:
            y = y + refs[idx][0:1, :]
            idx += 1
        if add is not None:
            y = y + refs[idx][...]
            idx += 1
        if relu:
            y = jnp.maximum(y, 0.0)
        refs[-1][...] = y

    return pl.pallas_call(
        body, grid=(grid,), in_specs=in_specs,
        out_specs=pl.BlockSpec((NBLK, m), lambda i: (i, 0)),
        out_shape=jax.ShapeDtypeStruct((n, m), F32),
    )(*args)


def _ent_tables(le, wT, b):
    n = le.shape[0]
    grid = n // NBLK

    def body(xr, wr, br, o1, o2):
        y = _dot(xr[...], wr[...]) + br[0:1, :]
        o1[...] = jnp.concatenate(
            [y[:, :LD], jnp.zeros((NBLK, LD), F32)], axis=1)
        o2[...] = y[:, LD:]

    return pl.pallas_call(
        body, grid=(grid,),
        in_specs=[pl.BlockSpec((NBLK, LD), lambda i: (i, 0)),
                  _full((LD, 3 * LD)), _full((8, 3 * LD))],
        out_specs=[pl.BlockSpec((NBLK, WROW), lambda i: (i, 0))] * 2,
        out_shape=[jax.ShapeDtypeStruct((n, WROW), F32)] * 2,
    )(le, wT, _bc8(b, 3 * LD))


def _edge_raw(gath, v64, S):
    npad = gath.shape[0]
    grid = npad // EBLK

    def body(pr, vr, sr, raw, pmax):
        h = _lrelu(pr[:, :LD]) * vr[0:1, :]
        r = _dot(h, sr[...])
        raw[...] = r
        pmax[...] = jnp.max(r, axis=0, keepdims=True)[None]

    return pl.pallas_call(
        body, grid=(grid,),
        in_specs=[pl.BlockSpec((EBLK, WROW), lambda i: (i, 0)),
                  _full((8, LD)), _full((LD, NHEAD))],
        out_specs=[pl.BlockSpec((EBLK, NHEAD), lambda i: (i, 0)),
                   pl.BlockSpec((1, 1, NHEAD), lambda i: (i, 0, 0))],
        out_shape=[jax.ShapeDtypeStruct((npad, NHEAD), F32),
                   jax.ShapeDtypeStruct((grid, 1, NHEAD), F32)],
    )(gath, _bc8(v64, LD), S)


def _edge_scale(raw, gath, gmax8, Sexp):
    npad = raw.shape[0]
    grid = npad // EBLK

    def body(rr, vr, gr, er, out):
        a = jnp.exp(rr[...] - gr[0:1, :])
        aggr = _dot(a, er[...]) * vr[:, LD:]
        out[...] = jnp.concatenate([aggr, a], axis=1)

    return pl.pallas_call(
        body, grid=(grid,),
        in_specs=[pl.BlockSpec((EBLK, NHEAD), lambda i: (i, 0)),
                  pl.BlockSpec((EBLK, WROW), lambda i: (i, 0)),
                  _full((8, NHEAD)), _full((NHEAD, LD))],
        out_specs=pl.BlockSpec((EBLK, VROW), lambda i: (i, 0)),
        out_shape=jax.ShapeDtypeStruct((npad, VROW), F32),
    )(raw, gath, gmax8, Sexp)


def _self_div(a0, a1):
    def body(r0, r1, out):
        s = r0[...] + r1[...]
        out[...] = s[:, :LD] / (s[:, LD:LD + 1] + 1e-16)

    return pl.pallas_call(
        body, grid=(10,),
        in_specs=[pl.BlockSpec((NBLK, WROW), lambda i: (i, 0))] * 2,
        out_specs=pl.BlockSpec((NBLK, LD), lambda i: (i, 0)),
        out_shape=jax.ShapeDtypeStruct((10000, LD), F32),
    )(a0, a1)


def _ent_combine(Tt, Th, CsHs, o0, le, WresT, bres, v64, S, Sexp, gmax8):
    nacc = len(o0)

    def body(*refs):
        ttr, thr, chr_ = refs[0], refs[1], refs[2]
        oas = refs[3:3 + nacc]
        ler, wr, brr, vr, sr, er, gmr, out = refs[3 + nacc:]
        A = ttr[:, :LD]
        B = thr[:, :LD]
        G = thr[:, LD:]
        cs = chr_[:, :LD]
        hs = chr_[:, LD:]
        h = _lrelu(A + B + cs) * vr[0:1, :]
        raw_s = _dot(h, sr[...])
        attn_s = jnp.exp(raw_s - gmr[0:1, :])
        vs = G + hs
        acc = oas[0][...]
        for oa in oas[1:]:
            acc = acc + oa[...]
        den = acc[:, LD:LD + NHEAD] + attn_s
        num = acc[:, :LD] + _dot(attn_s, er[...]) * vs
        o = num / (_dot(den, er[...]) + 1e-38)
        o = o + _dot(ler[...], wr[...]) + brr[0:1, :]
        out[...] = jnp.maximum(o, 0.0)

    blk = lambda w: pl.BlockSpec((NBLK, w), lambda i: (i, 0))
    return pl.pallas_call(
        body, grid=(10,),
        in_specs=[blk(WROW), blk(WROW), blk(WROW)]
        + [blk(VROW)] * len(o0)
        + [blk(LD), _full((LD, LD)), _full((8, LD)), _full((8, LD)),
           _full((LD, NHEAD)), _full((NHEAD, LD)), _full((8, NHEAD))],
        out_specs=blk(LD),
        out_shape=jax.ShapeDtypeStruct((10000, LD), F32),
    )(Tt, Th, CsHs, *o0, le, WresT, _bc8(bres, LD), _bc8(v64, LD),
      S, Sexp, gmax8)


def _rel_dense(lr16, W1hT, W1tT, battn, WaT, ba, bin16, n8, Oh, Ot, Ob,
               v64, S, Sexp):
    def body(lrr, whr, wtr, bar, war, bagr, binr, n8r, ohr, otr, obr,
             vr, sr, er, out):
        lrv = lrr[...]
        Ah = _dot(lrv, whr[...]) + bar[0:1, :]
        Bt = _dot(lrv, wtr[...])
        V = _dot(lrv, war[...]) + bagr[0:1, :]
        n1 = n8r[:, 0:1]
        pre = _dot(ohr[...], Ah) + _dot(otr[...], Bt)
        raw = _dot(_lrelu(pre) * vr[0:1, :], sr[...]) + _dot(obr[...], binr[...])
        gmax = jnp.max(raw, axis=0, keepdims=True)
        E = n1 * jnp.exp(raw - gmax)
        dnums = (((0,), (0,)), ((), ()))
        den = lax.dot_general(ohr[...], E, dnums, preferred_element_type=F32)
        Vc = _dot(otr[...], V)
        num = lax.dot_general(ohr[...], _dot(E, er[...]) * Vc, dnums,
                              preferred_element_type=F32)
        out[...] = num / (_dot(den, er[...]) + 1e-38)

    return pl.pallas_call(
        body, grid=(1,),
        in_specs=[_full((16, LD)), _full((LD, LD)), _full((LD, LD)),
                  _full((8, LD)), _full((LD, LD)), _full((8, LD)),
                  _full((16, NHEAD)),
                  _full((1024, 8)),
                  _full((1024, 16)), _full((1024, 16)), _full((1024, 16)),
                  _full((8, LD)), _full((LD, NHEAD)), _full((NHEAD, LD))],
        out_specs=_full((16, LD)),
        out_shape=jax.ShapeDtypeStruct((16, LD), F32),
    )(lr16, W1hT, W1tT, _bc8(battn, LD), WaT, _bc8(ba, LD), bin16, n8,
      Oh, Ot, Ob, _bc8(v64, LD), S, Sexp)


def kernel(emb_ent, emb_rel, triplets, relation_triplets, params):
    S = (jnp.arange(LD)[:, None] // DH == jnp.arange(NHEAD)[None, :]).astype(F32)
    Sexp = S.T

    n_tri = triplets.shape[0]
    npad_e = _pad_rows(n_tri, NW * CHG * 2)
    pe = npad_e - n_tri
    h_g = jnp.pad(triplets[:, 0], (0, pe))
    r_g = jnp.pad(triplets[:, 1], (0, pe))
    t_g = jnp.pad(triplets[:, 2], (0, pe))
    t_s = jnp.pad(triplets[:, 2], (0, pe), constant_values=10000)

    n_rt = relation_triplets.shape[0]
    npad_r = _pad_rows(n_rt, EBLK)
    cidx = (relation_triplets[:, 0] * 100 + relation_triplets[:, 1] * 10
            + relation_triplets[:, 2])
    cidx = jnp.pad(cidx, (0, npad_r - n_rt), constant_values=1000)
    h8 = jnp.broadcast_to((cidx // 100)[:, None], (npad_r, 8))
    tb8 = jnp.broadcast_to((cidx % 100)[:, None], (npad_r, 8))

    c1024 = jnp.arange(1024)
    valid = (c1024 < 1000)[:, None]
    Oh = ((c1024[:, None] // 100 == jnp.arange(16)[None, :]) & valid).astype(F32)
    Ot = (((c1024[:, None] // 10) % 10 == jnp.arange(16)[None, :]) & valid).astype(F32)
    Ob = ((c1024[:, None] % 10 == jnp.arange(16)[None, :]) & valid).astype(F32)

    le = _lin(emb_ent, params['ent_proj1_w'].T, params['ent_proj1_b'])
    lr = _lin(emb_rel, params['rel_proj1_w'].T, params['rel_proj1_b'])

    N16 = _tc_hist(h8, tb8)
    n1024 = jnp.pad(N16[:10, :100].reshape(-1), (0, 24))
    n8 = jnp.broadcast_to(n1024[:, None], (1024, 8))
    for i in range(NLAYER):
        p = params['rel_layers'][i]
        W = p['attn_proj_w']
        lr16 = jnp.pad(lr[:10], ((0, 6), (0, 0)))
        bin16 = jnp.pad(p['attn_bin'].reshape(NBIN, NHEAD), ((0, 6), (0, 0)))
        out16 = _rel_dense(lr16, W[:, :LD].T, W[:, LD:].T, p['attn_proj_b'],
                           p['aggr_proj_w'].T, p['aggr_proj_b'], bin16,
                           n8, Oh, Ot, Ob, p['attn_vec'], S, Sexp)
        out_full = jnp.zeros((10000, LD), F32).at[:10].set(out16[:10])
        pr = params['res_rel'][i]
        lr = _lin(lr, pr['w'].T, pr['b'], add=out_full, relu=True)

    table = jnp.concatenate(
        [lr, jnp.ones((10000, 1), F32), jnp.zeros((10000, WROW - LD - 1), F32)],
        axis=1)
    sacc = _sc_gather_scatter(table, r_g, t_s, npad_e)
    self_rel = _self_div(sacc[0, :10000], sacc[1, :10000])

    for i in range(NLAYER):
        p = params['ent_layers'][i]
        W = p['attn_proj_w']
        Wa = p['aggr_proj_w']
        bb = jnp.concatenate([p['attn_proj_b'], jnp.zeros((LD,), F32),
                              p['aggr_proj_b']])
        wcat = jnp.concatenate(
            [W[:, :LD].T, W[:, LD:2 * LD].T, Wa[:, :LD].T], axis=1)
        Tt, Th = _ent_tables(le, wcat, bb)
        wr_cat = jnp.concatenate([W[:, 2 * LD:].T, Wa[:, LD:].T], axis=1)
        Tr = _lin(lr, wr_cat)
        CsHs = _lin(self_rel, wr_cat)

        gath = _sc_gather_sum([Tt, Th, Tr], [t_g, h_g, r_g], npad_e)
        raw, pmax = _edge_raw(gath, p['attn_vec'], S)
        gmax8 = jnp.broadcast_to(
            jnp.max(pmax, axis=(0, 1)).reshape(1, NHEAD), (8, NHEAD))
        vals = _edge_scale(raw, gath, gmax8, Sexp)
        oacc = _sc_scatter(t_s, vals, npad_e)
        oaccs = [oacc[0, :10000], oacc[1, :10000]]

        pr = params['res_ent'][i]
        le = _ent_combine(Tt, Th, CsHs, oaccs, le, pr['w'].T, pr['b'],
                          p['attn_vec'], S, Sexp, gmax8)

    out_ent = _lin(le, params['ent_proj2_w'].T, params['ent_proj2_b'])
    out_rel = _lin(lr, params['rel_proj2_w'].T, params['rel_proj2_b'])
    return out_ent, out_rel
